# baseline (speedup 1.0000x reference)
import functools

import jax
import jax.numpy as jnp
from jax import lax
from jax.experimental import pallas as pl
from jax.experimental.pallas import tpu as pltpu
from jax.experimental.pallas import tpu_sc as plsc

DIM = 128
V_SIZE = 10000
C_SIZE = 40000
N_ITER = 2

NC = 2
NS = 16

NE = 80000
B_E = 64
K_PIPE = 4
GRAN = NS * B_E * K_PIPE
CH = 10000
ACC_R = 10240
WPT = ACC_R // NS
ZR = 16


def _sc_aggregate_body(cfg, table, esrc, edst, params, out, src_v, dst_v,
                       sidx, idx, rows, zero_v, par_v, acc, semg, sems):
    n_dst, nchunk, capb = cfg
    e_tile = capb * B_E
    c = lax.axis_index("c")
    s = lax.axis_index("s")
    pltpu.sync_copy(params, par_v)
    pv = par_v[pl.ds(0, 16)]
    gath = pv[0]
    ebase = c * (e_tile * NS) + s * e_tile
    pltpu.sync_copy(esrc.at[pl.ds(ebase, e_tile)], src_v)
    pltpu.sync_copy(edst.at[pl.ds(ebase, e_tile)], dst_v)
    zeros16 = jnp.zeros((16,), jnp.float32)
    ones16 = jnp.ones((16,), jnp.float32)
    iota16 = jnp.arange(16, dtype=jnp.int32)

    def _zrow(i, _):
        for u in range(8):
            zero_v[i, pl.ds(u * 16, 16)] = zeros16
        return 0

    lax.fori_loop(0, ZR, _zrow, 0)

    @pl.when(gath == 0)
    def _():
        def _orow(i, _):
            for t in range(K_PIPE):
                for u in range(8):
                    rows[t][i, pl.ds(u * 16, 16)] = ones16
            return 0

        lax.fori_loop(0, B_E, _orow, 0)

    for k in range(nchunk):
        base = k * CH
        b0 = jnp.where(c == 0, pv[1 + k], pv[6 + k])
        b1 = jnp.where(c == 0, pv[2 + k], pv[7 + k])
        for m in range(WPT // ZR):
            pltpu.sync_copy(zero_v, acc.at[pl.ds(s * WPT + m * ZR, ZR)])
        plsc.subcore_barrier()

        def _group(q, _):
            goff = (b0 + q * K_PIPE) * B_E
            for t in range(K_PIPE):
                off = goff + t * B_E
                for i in range(B_E // 16):
                    dv = dst_v[pl.ds(off + i * 16, 16)]
                    lo = dv - base
                    ok = (lo >= 0) & (lo < CH)
                    idx[t][pl.ds(i * 16, 16)] = jnp.where(ok, lo,
                                                          CH + i * 16 + iota16)
                    sidx[t][pl.ds(i * 16, 16)] = src_v[pl.ds(off + i * 16, 16)]

            @pl.when(gath > 0)
            def _():
                waits = [pltpu.async_copy(table.at[sidx[t]], rows[t], semg[t])
                         for t in range(K_PIPE)]
                for t in range(K_PIPE):
                    waits[t].wait()

            drains = [pltpu.async_copy(rows[t], acc.at[idx[t]], sems[t],
                                       add=True)
                      for t in range(K_PIPE)]
            for t in range(K_PIPE):
                drains[t].wait()
            return 0

        lax.fori_loop(0, (b1 - b0) // K_PIPE, _group, 0)
        plsc.subcore_barrier()
        @pl.when(s < NS - 1)
        def _():
            pltpu.sync_copy(acc.at[pl.ds(s * WPT, WPT)],
                            out.at[c, pl.ds(base + s * WPT, WPT)])

        @pl.when(s == NS - 1)
        def _():
            pltpu.sync_copy(acc.at[pl.ds((NS - 1) * WPT, CH - (NS - 1) * WPT)],
                            out.at[c, pl.ds(base + (NS - 1) * WPT,
                                            CH - (NS - 1) * WPT)])

        plsc.subcore_barrier()


def _make_sc_aggregate(n_dst, nchunk, capb):
    mesh = plsc.VectorSubcoreMesh(core_axis_name="c", subcore_axis_name="s",
                                  num_cores=NC, num_subcores=NS)
    e_tile = capb * B_E
    return pl.kernel(
        functools.partial(_sc_aggregate_body, (n_dst, nchunk, capb)),
        out_type=jax.ShapeDtypeStruct((NC, n_dst, DIM), jnp.float32),
        mesh=mesh,
        scratch_types=[
            pltpu.VMEM((e_tile,), jnp.int32),
            pltpu.VMEM((e_tile,), jnp.int32),
            [pltpu.VMEM((B_E,), jnp.int32) for _ in range(K_PIPE)],
            [pltpu.VMEM((B_E,), jnp.int32) for _ in range(K_PIPE)],
            [pltpu.VMEM((B_E, DIM), jnp.float32) for _ in range(K_PIPE)],
            pltpu.VMEM((ZR, DIM), jnp.float32),
            pltpu.VMEM((16,), jnp.int32),
            pltpu.VMEM_SHARED((ACC_R, DIM), jnp.float32),
            [pltpu.SemaphoreType.DMA for _ in range(K_PIPE)],
            [pltpu.SemaphoreType.DMA for _ in range(K_PIPE)],
        ],
    )


def _mlp_pair_body(x_ref, w_ref, b_ref, o_ref):
    x = x_ref[...]
    for t in range(2):
        h = jnp.dot(x, w_ref[t, 0], preferred_element_type=jnp.float32)
        h = jnp.maximum(h + b_ref[t, 0], 0.0)
        o_ref[t] = jnp.dot(h, w_ref[t, 1],
                           preferred_element_type=jnp.float32) + b_ref[t, 1]


def _mlp_pair(x, w2, b2, blk):
    n = x.shape[0]
    return pl.pallas_call(
        _mlp_pair_body,
        grid=(n // blk,),
        in_specs=[
            pl.BlockSpec((blk, DIM), lambda i: (i, 0)),
            pl.BlockSpec((2, 2, DIM, DIM), lambda i: (0, 0, 0, 0)),
            pl.BlockSpec((2, 2, DIM), lambda i: (0, 0, 0)),
        ],
        out_specs=pl.BlockSpec((2, blk, DIM), lambda i: (0, i, 0)),
        out_shape=jax.ShapeDtypeStruct((2, n, DIM), jnp.float32),
    )(x, w2, b2)


def _gru_body(aggr_ref, invp_ref, invn_ref, h_ref, wih_ref, whh_ref,
              bih_ref, bhh_ref, o_ref):
    xp = aggr_ref[0] * invp_ref[...]
    xn = aggr_ref[1] * invn_ref[...]
    gi = (jnp.dot(xp, wih_ref[0:DIM], preferred_element_type=jnp.float32)
          + jnp.dot(xn, wih_ref[DIM:2 * DIM],
                    preferred_element_type=jnp.float32) + bih_ref[...])
    h = h_ref[...]
    gh = jnp.dot(h, whh_ref[...], preferred_element_type=jnp.float32) + bhh_ref[...]
    r = jax.nn.sigmoid(gi[:, 0:DIM] + gh[:, 0:DIM])
    z = jax.nn.sigmoid(gi[:, DIM:2 * DIM] + gh[:, DIM:2 * DIM])
    nn = jnp.tanh(gi[:, 2 * DIM:] + r * gh[:, 2 * DIM:])
    o_ref[...] = (1.0 - z) * nn + z * h


def _gru(aggr, invp, invn, h, wih_t, whh_t, bih, bhh, blk):
    n = h.shape[0]
    return pl.pallas_call(
        _gru_body,
        grid=(n // blk,),
        in_specs=[
            pl.BlockSpec((2, blk, DIM), lambda i: (0, i, 0)),
            pl.BlockSpec((blk, 1), lambda i: (i, 0)),
            pl.BlockSpec((blk, 1), lambda i: (i, 0)),
            pl.BlockSpec((blk, DIM), lambda i: (i, 0)),
            pl.BlockSpec((2 * DIM, 3 * DIM), lambda i: (0, 0)),
            pl.BlockSpec((DIM, 3 * DIM), lambda i: (0, 0)),
            pl.BlockSpec((3 * DIM,), lambda i: (0,)),
            pl.BlockSpec((3 * DIM,), lambda i: (0,)),
        ],
        out_specs=pl.BlockSpec((blk, DIM), lambda i: (i, 0)),
        out_shape=jax.ShapeDtypeStruct((n, DIM), jnp.float32),
    )(aggr, invp, invn, h, wih_t, whh_t, bih, bhh)


def _bucket_edges(src, dst, n_dst, nchunk, capb):
    cid = jnp.minimum(dst // CH, nchunk - 1)
    order = jnp.argsort(cid, stable=True)
    s_src = src[order]
    s_dst = dst[order]
    s_cid = cid[order]
    counts = jnp.zeros((nchunk,), jnp.int32).at[cid].add(1)
    lk = ((counts + GRAN - 1) // GRAN) * GRAN
    cum = jnp.concatenate([jnp.zeros((1,), jnp.int32), jnp.cumsum(lk)])
    cstart = jnp.concatenate([jnp.zeros((1,), jnp.int32),
                              jnp.cumsum(counts)])[:nchunk]
    w = jnp.arange(NE, dtype=jnp.int32) - cstart[s_cid]
    pt = jnp.maximum(lk // NS, 1)[s_cid]
    tile = w // pt
    slot = (cum[:nchunk] // NS)[s_cid] + w % pt
    tgt = tile * (capb * B_E) + slot
    cap = NS * capb * B_E
    src_pad = jnp.zeros((cap,), jnp.int32).at[tgt].set(s_src)
    dst_pad = jnp.full((cap,), n_dst, jnp.int32).at[tgt].set(s_dst)
    cb = cum // (NS * B_E)
    return src_pad, dst_pad, cb


def kernel(v_init, c_init, mlp_W, mlp_b, gru_Wih, gru_Whh, gru_bih, gru_bhh,
           v_edge_index, c_edge_index, p_edge_index, n_edge_index):
    f32 = jnp.float32
    v_edge_index = v_edge_index.astype(jnp.int32)
    c_edge_index = c_edge_index.astype(jnp.int32)
    p_edge_index = p_edge_index.astype(jnp.int32)
    n_edge_index = n_edge_index.astype(jnp.int32)

    pv = v_edge_index[p_edge_index]
    pc = c_edge_index[p_edge_index]
    nv = v_edge_index[n_edge_index]
    nc = c_edge_index[n_edge_index]

    capb_c = (NE + GRAN - 1) // GRAN * K_PIPE + (4 - 1) * K_PIPE
    capb_v = (NE + GRAN - 1) // GRAN * K_PIPE

    psrc_c, pdst_c, cb_pc = _bucket_edges(pv, pc, C_SIZE, 4, capb_c)
    nsrc_c, ndst_c, cb_nc = _bucket_edges(nv, nc, C_SIZE, 4, capb_c)
    esrc_c = jnp.concatenate([psrc_c, nsrc_c + V_SIZE])
    edst_c = jnp.concatenate([pdst_c, ndst_c])
    psrc_v, pdst_v, cb_pv = _bucket_edges(pc, pv, V_SIZE, 1, capb_v)
    nsrc_v, ndst_v, cb_nv = _bucket_edges(nc, nv, V_SIZE, 1, capb_v)
    esrc_v = jnp.concatenate([psrc_v, nsrc_v + C_SIZE])
    edst_v = jnp.concatenate([pdst_v, ndst_v])

    def pack_par(mode, cb0, cb1):
        p = jnp.zeros((16,), jnp.int32)
        p = p.at[0].set(mode)
        p = lax.dynamic_update_slice(p, cb0.astype(jnp.int32), (1,))
        p = lax.dynamic_update_slice(p, cb1.astype(jnp.int32), (6,))
        return p

    par_c_deg = pack_par(0, cb_pc, cb_nc)
    par_c_agg = pack_par(1, cb_pc, cb_nc)
    par_v_deg = pack_par(0, cb_pv, cb_nv)
    par_v_agg = pack_par(1, cb_pv, cb_nv)

    sc_v2c = _make_sc_aggregate(C_SIZE, 4, capb_c)
    sc_c2v = _make_sc_aggregate(V_SIZE, 1, capb_v)

    wih_t = jnp.transpose(gru_Wih, (0, 2, 1)).astype(f32)
    whh_t = jnp.transpose(gru_Whh, (0, 2, 1)).astype(f32)

    init_norm = jnp.sqrt(jnp.asarray(DIM, f32))
    v_emb = jnp.broadcast_to(v_init / init_norm, (V_SIZE, DIM)).astype(f32)
    c_emb = jnp.broadcast_to(c_init / init_norm, (C_SIZE, DIM)).astype(f32)

    inv_pc = inv_nc = inv_pv = inv_nv = None
    for it in range(N_ITER):
        t_v = _mlp_pair(v_emb, mlp_W[0:2], mlp_b[0:2], 2000)
        tvf = t_v.reshape(2 * V_SIZE, DIM)
        if it == 0:
            deg_c = sc_v2c(tvf, esrc_c, edst_c, par_c_deg)
            inv_pc = (1.0 / jnp.maximum(deg_c[0, :, 0], 1.0)).reshape(C_SIZE, 1)
            inv_nc = (1.0 / jnp.maximum(deg_c[1, :, 0], 1.0)).reshape(C_SIZE, 1)
        aggr_c = sc_v2c(tvf, esrc_c, edst_c, par_c_agg)
        c_emb = _gru(aggr_c, inv_pc, inv_nc, c_emb,
                     wih_t[0], whh_t[0], gru_bih[0], gru_bhh[0], 2000)
        t_c = _mlp_pair(c_emb, mlp_W[2:4], mlp_b[2:4], 2000)
        tcf = t_c.reshape(2 * C_SIZE, DIM)
        if it == 0:
            deg_v = sc_c2v(tcf, esrc_v, edst_v, par_v_deg)
            inv_pv = (1.0 / jnp.maximum(deg_v[0, :, 0], 1.0)).reshape(V_SIZE, 1)
            inv_nv = (1.0 / jnp.maximum(deg_v[1, :, 0], 1.0)).reshape(V_SIZE, 1)
        aggr_v = sc_c2v(tcf, esrc_v, edst_v, par_v_agg)
        v_emb = _gru(aggr_v, inv_pv, inv_nv, v_emb,
                     wih_t[1], whh_t[1], gru_bih[1], gru_bhh[1], 2000)
    return v_emb


# speedup vs baseline: 1.1303x; 1.1303x over previous
import functools

import jax
import jax.numpy as jnp
from jax import lax
from jax.experimental import pallas as pl
from jax.experimental.pallas import tpu as pltpu
from jax.experimental.pallas import tpu_sc as plsc

DIM = 128
V_SIZE = 10000
C_SIZE = 40000
N_ITER = 2

NC = 2
NS = 16

NE = 80000
B_E = 64
K_PIPE = 4
GRAN = NS * B_E * K_PIPE
CH = 10000
ACC_R = 10240
WPT = ACC_R // NS
ZR = 16


def _sc_aggregate_body(cfg, table, esrc, edst, params, out, src_v, dst_v,
                       sidx, idx, rows, zero_v, par_v, acc, semg, sems):
    n_dst, nchunk, capb = cfg
    e_tile = capb * B_E
    c = lax.axis_index("c")
    s = lax.axis_index("s")
    pltpu.sync_copy(params, par_v)
    pv = par_v[pl.ds(0, 16)]
    gath = pv[0]
    ebase = c * (e_tile * NS) + s * e_tile
    pltpu.sync_copy(esrc.at[pl.ds(ebase, e_tile)], src_v)
    pltpu.sync_copy(edst.at[pl.ds(ebase, e_tile)], dst_v)
    zeros16 = jnp.zeros((16,), jnp.float32)
    ones16 = jnp.ones((16,), jnp.float32)
    iota16 = jnp.arange(16, dtype=jnp.int32)

    def _zrow(i, _):
        for u in range(8):
            zero_v[i, pl.ds(u * 16, 16)] = zeros16
        return 0

    lax.fori_loop(0, ZR, _zrow, 0)

    @pl.when(gath == 0)
    def _():
        def _orow(i, _):
            for t in range(K_PIPE):
                for u in range(8):
                    rows[t][i, pl.ds(u * 16, 16)] = ones16
            return 0

        lax.fori_loop(0, B_E, _orow, 0)

    for k in range(nchunk):
        base = k * CH
        b0 = jnp.where(c == 0, pv[1 + k], pv[6 + k])
        b1 = jnp.where(c == 0, pv[2 + k], pv[7 + k])
        for m in range(WPT // ZR):
            pltpu.sync_copy(zero_v, acc.at[pl.ds(s * WPT + m * ZR, ZR)])
        plsc.subcore_barrier()

        def _group(q, _):
            goff = (b0 + q * K_PIPE) * B_E
            for t in range(K_PIPE):
                off = goff + t * B_E
                for i in range(B_E // 16):
                    dv = dst_v[pl.ds(off + i * 16, 16)]
                    lo = dv - base
                    ok = (lo >= 0) & (lo < CH)
                    idx[t][pl.ds(i * 16, 16)] = jnp.where(ok, lo,
                                                          CH + i * 16 + iota16)
                    sidx[t][pl.ds(i * 16, 16)] = src_v[pl.ds(off + i * 16, 16)]

            @pl.when(gath > 0)
            def _():
                waits = [pltpu.async_copy(table.at[sidx[t]], rows[t], semg[t])
                         for t in range(K_PIPE)]
                for t in range(K_PIPE):
                    waits[t].wait()

            drains = [pltpu.async_copy(rows[t], acc.at[idx[t]], sems[t],
                                       add=True)
                      for t in range(K_PIPE)]
            for t in range(K_PIPE):
                drains[t].wait()
            return 0

        lax.fori_loop(0, (b1 - b0) // K_PIPE, _group, 0)
        plsc.subcore_barrier()
        @pl.when(s < NS - 1)
        def _():
            pltpu.sync_copy(acc.at[pl.ds(s * WPT, WPT)],
                            out.at[c, pl.ds(base + s * WPT, WPT)])

        @pl.when(s == NS - 1)
        def _():
            pltpu.sync_copy(acc.at[pl.ds((NS - 1) * WPT, CH - (NS - 1) * WPT)],
                            out.at[c, pl.ds(base + (NS - 1) * WPT,
                                            CH - (NS - 1) * WPT)])

        plsc.subcore_barrier()


def _make_sc_aggregate(n_dst, nchunk, capb):
    mesh = plsc.VectorSubcoreMesh(core_axis_name="c", subcore_axis_name="s",
                                  num_cores=NC, num_subcores=NS)
    e_tile = capb * B_E
    return pl.kernel(
        functools.partial(_sc_aggregate_body, (n_dst, nchunk, capb)),
        out_type=jax.ShapeDtypeStruct((NC, n_dst, DIM), jnp.float32),
        mesh=mesh,
        scratch_types=[
            pltpu.VMEM((e_tile,), jnp.int32),
            pltpu.VMEM((e_tile,), jnp.int32),
            [pltpu.VMEM((B_E,), jnp.int32) for _ in range(K_PIPE)],
            [pltpu.VMEM((B_E,), jnp.int32) for _ in range(K_PIPE)],
            [pltpu.VMEM((B_E, DIM), jnp.float32) for _ in range(K_PIPE)],
            pltpu.VMEM((ZR, DIM), jnp.float32),
            pltpu.VMEM((16,), jnp.int32),
            pltpu.VMEM_SHARED((ACC_R, DIM), jnp.float32),
            [pltpu.SemaphoreType.DMA for _ in range(K_PIPE)],
            [pltpu.SemaphoreType.DMA for _ in range(K_PIPE)],
        ],
    )


def _mlp_pair_body(x_ref, w_ref, b_ref, o_ref):
    x = x_ref[...]
    for t in range(2):
        h = jnp.dot(x, w_ref[t, 0], preferred_element_type=jnp.float32)
        h = jnp.maximum(h + b_ref[t, 0], 0.0)
        o_ref[t] = jnp.dot(h, w_ref[t, 1],
                           preferred_element_type=jnp.float32) + b_ref[t, 1]


def _mlp_pair(x, w2, b2, blk):
    n = x.shape[0]
    return pl.pallas_call(
        _mlp_pair_body,
        grid=(n // blk,),
        in_specs=[
            pl.BlockSpec((blk, DIM), lambda i: (i, 0)),
            pl.BlockSpec((2, 2, DIM, DIM), lambda i: (0, 0, 0, 0)),
            pl.BlockSpec((2, 2, DIM), lambda i: (0, 0, 0)),
        ],
        out_specs=pl.BlockSpec((2, blk, DIM), lambda i: (0, i, 0)),
        out_shape=jax.ShapeDtypeStruct((2, n, DIM), jnp.float32),
    )(x, w2, b2)


def _gru_body(aggr_ref, invp_ref, invn_ref, h_ref, wih_ref, whh_ref,
              bih_ref, bhh_ref, o_ref):
    xp = aggr_ref[0] * invp_ref[...]
    xn = aggr_ref[1] * invn_ref[...]
    gi = (jnp.dot(xp, wih_ref[0:DIM], preferred_element_type=jnp.float32)
          + jnp.dot(xn, wih_ref[DIM:2 * DIM],
                    preferred_element_type=jnp.float32) + bih_ref[...])
    h = h_ref[...]
    gh = jnp.dot(h, whh_ref[...], preferred_element_type=jnp.float32) + bhh_ref[...]
    r = jax.nn.sigmoid(gi[:, 0:DIM] + gh[:, 0:DIM])
    z = jax.nn.sigmoid(gi[:, DIM:2 * DIM] + gh[:, DIM:2 * DIM])
    nn = jnp.tanh(gi[:, 2 * DIM:] + r * gh[:, 2 * DIM:])
    o_ref[...] = (1.0 - z) * nn + z * h


def _gru(aggr, invp, invn, h, wih_t, whh_t, bih, bhh, blk):
    n = h.shape[0]
    return pl.pallas_call(
        _gru_body,
        grid=(n // blk,),
        in_specs=[
            pl.BlockSpec((2, blk, DIM), lambda i: (0, i, 0)),
            pl.BlockSpec((blk, 1), lambda i: (i, 0)),
            pl.BlockSpec((blk, 1), lambda i: (i, 0)),
            pl.BlockSpec((blk, DIM), lambda i: (i, 0)),
            pl.BlockSpec((2 * DIM, 3 * DIM), lambda i: (0, 0)),
            pl.BlockSpec((DIM, 3 * DIM), lambda i: (0, 0)),
            pl.BlockSpec((3 * DIM,), lambda i: (0,)),
            pl.BlockSpec((3 * DIM,), lambda i: (0,)),
        ],
        out_specs=pl.BlockSpec((blk, DIM), lambda i: (i, 0)),
        out_shape=jax.ShapeDtypeStruct((n, DIM), jnp.float32),
    )(aggr, invp, invn, h, wih_t, whh_t, bih, bhh)


def _bucket_edges(src, dst, n_dst, nchunk, capb):
    cid = jnp.minimum(dst // CH, nchunk - 1)
    onek = (cid[None, :] == jnp.arange(nchunk, dtype=jnp.int32)[:, None])
    ranks = jnp.cumsum(onek.astype(jnp.int32), axis=1)
    w = jnp.take_along_axis(ranks, cid[None, :], 0)[0] - 1
    counts = ranks[:, -1]
    lk = ((counts + GRAN - 1) // GRAN) * GRAN
    cum = jnp.concatenate([jnp.zeros((1,), jnp.int32), jnp.cumsum(lk)])
    pt = jnp.maximum(lk // NS, 1)[cid]
    tile = w // pt
    slot = (cum[:nchunk] // NS)[cid] + w % pt
    tgt = tile * (capb * B_E) + slot
    cap = NS * capb * B_E
    src_pad = jnp.zeros((cap,), jnp.int32).at[tgt].set(src)
    dst_pad = jnp.full((cap,), n_dst, jnp.int32).at[tgt].set(dst)
    cb = cum // (NS * B_E)
    return src_pad, dst_pad, cb


def kernel(v_init, c_init, mlp_W, mlp_b, gru_Wih, gru_Whh, gru_bih, gru_bhh,
           v_edge_index, c_edge_index, p_edge_index, n_edge_index):
    f32 = jnp.float32
    v_edge_index = v_edge_index.astype(jnp.int32)
    c_edge_index = c_edge_index.astype(jnp.int32)
    p_edge_index = p_edge_index.astype(jnp.int32)
    n_edge_index = n_edge_index.astype(jnp.int32)

    pv = v_edge_index[p_edge_index]
    pc = c_edge_index[p_edge_index]
    nv = v_edge_index[n_edge_index]
    nc = c_edge_index[n_edge_index]

    capb_c = (NE + GRAN - 1) // GRAN * K_PIPE + (4 - 1) * K_PIPE
    capb_v = (NE + GRAN - 1) // GRAN * K_PIPE

    psrc_c, pdst_c, cb_pc = _bucket_edges(pv, pc, C_SIZE, 4, capb_c)
    nsrc_c, ndst_c, cb_nc = _bucket_edges(nv, nc, C_SIZE, 4, capb_c)
    esrc_c = jnp.concatenate([psrc_c, nsrc_c + V_SIZE])
    edst_c = jnp.concatenate([pdst_c, ndst_c])
    psrc_v, pdst_v, cb_pv = _bucket_edges(pc, pv, V_SIZE, 1, capb_v)
    nsrc_v, ndst_v, cb_nv = _bucket_edges(nc, nv, V_SIZE, 1, capb_v)
    esrc_v = jnp.concatenate([psrc_v, nsrc_v + C_SIZE])
    edst_v = jnp.concatenate([pdst_v, ndst_v])

    def pack_par(mode, cb0, cb1):
        p = jnp.zeros((16,), jnp.int32)
        p = p.at[0].set(mode)
        p = lax.dynamic_update_slice(p, cb0.astype(jnp.int32), (1,))
        p = lax.dynamic_update_slice(p, cb1.astype(jnp.int32), (6,))
        return p

    par_c_deg = pack_par(0, cb_pc, cb_nc)
    par_c_agg = pack_par(1, cb_pc, cb_nc)
    par_v_deg = pack_par(0, cb_pv, cb_nv)
    par_v_agg = pack_par(1, cb_pv, cb_nv)

    sc_v2c = _make_sc_aggregate(C_SIZE, 4, capb_c)
    sc_c2v = _make_sc_aggregate(V_SIZE, 1, capb_v)

    wih_t = jnp.transpose(gru_Wih, (0, 2, 1)).astype(f32)
    whh_t = jnp.transpose(gru_Whh, (0, 2, 1)).astype(f32)

    init_norm = jnp.sqrt(jnp.asarray(DIM, f32))
    v_emb = jnp.broadcast_to(v_init / init_norm, (V_SIZE, DIM)).astype(f32)
    c_emb = jnp.broadcast_to(c_init / init_norm, (C_SIZE, DIM)).astype(f32)

    inv_pc = inv_nc = inv_pv = inv_nv = None
    for it in range(N_ITER):
        t_v = _mlp_pair(v_emb, mlp_W[0:2], mlp_b[0:2], 2000)
        tvf = t_v.reshape(2 * V_SIZE, DIM)
        if it == 0:
            deg_c = sc_v2c(tvf, esrc_c, edst_c, par_c_deg)
            inv_pc = (1.0 / jnp.maximum(deg_c[0, :, 0], 1.0)).reshape(C_SIZE, 1)
            inv_nc = (1.0 / jnp.maximum(deg_c[1, :, 0], 1.0)).reshape(C_SIZE, 1)
        aggr_c = sc_v2c(tvf, esrc_c, edst_c, par_c_agg)
        c_emb = _gru(aggr_c, inv_pc, inv_nc, c_emb,
                     wih_t[0], whh_t[0], gru_bih[0], gru_bhh[0], 2000)
        t_c = _mlp_pair(c_emb, mlp_W[2:4], mlp_b[2:4], 2000)
        tcf = t_c.reshape(2 * C_SIZE, DIM)
        if it == 0:
            deg_v = sc_c2v(tcf, esrc_v, edst_v, par_v_deg)
            inv_pv = (1.0 / jnp.maximum(deg_v[0, :, 0], 1.0)).reshape(V_SIZE, 1)
            inv_nv = (1.0 / jnp.maximum(deg_v[1, :, 0], 1.0)).reshape(V_SIZE, 1)
        aggr_v = sc_c2v(tcf, esrc_v, edst_v, par_v_agg)
        v_emb = _gru(aggr_v, inv_pv, inv_nv, v_emb,
                     wih_t[1], whh_t[1], gru_bih[1], gru_bhh[1], 2000)
    return v_emb


# speedup vs baseline: 2.2250x; 1.9685x over previous
import functools

import jax
import jax.numpy as jnp
from jax import lax
from jax.experimental import pallas as pl
from jax.experimental.pallas import tpu as pltpu
from jax.experimental.pallas import tpu_sc as plsc

DIM = 128
V_SIZE = 10000
C_SIZE = 40000
N_ITER = 2

NC = 2
NS = 16

NE = 80000
B_E = 64
K_PIPE = 4
GRAN = NS * B_E * K_PIPE
CH = 10000
ACC_R = 10240
WPT = ACC_R // NS
ZR = 16


def _sc_aggregate_body(cfg, table, esrc, edst, params, out, src_v, dst_v,
                       sidx, idx, rows, zero_v, par_v, acc, semg, sems):
    n_dst, nchunk, capb = cfg
    e_tile = capb * B_E
    c = lax.axis_index("c")
    s = lax.axis_index("s")
    pltpu.sync_copy(params, par_v)
    pv = par_v[pl.ds(0, 16)]
    gath = pv[0]
    ebase = c * (e_tile * NS) + s * e_tile
    pltpu.sync_copy(esrc.at[pl.ds(ebase, e_tile)], src_v)
    pltpu.sync_copy(edst.at[pl.ds(ebase, e_tile)], dst_v)
    zeros16 = jnp.zeros((16,), jnp.float32)
    ones16 = jnp.ones((16,), jnp.float32)
    iota16 = jnp.arange(16, dtype=jnp.int32)

    def _zrow(i, _):
        for u in range(8):
            zero_v[i, pl.ds(u * 16, 16)] = zeros16
        return 0

    lax.fori_loop(0, ZR, _zrow, 0)

    @pl.when(gath == 0)
    def _():
        def _orow(i, _):
            for t in range(K_PIPE):
                for u in range(8):
                    rows[t][i, pl.ds(u * 16, 16)] = ones16
            return 0

        lax.fori_loop(0, B_E, _orow, 0)

    for k in range(nchunk):
        base = k * CH
        b0 = jnp.where(c == 0, pv[1 + k], pv[6 + k])
        b1 = jnp.where(c == 0, pv[2 + k], pv[7 + k])
        for m in range(WPT // ZR):
            pltpu.sync_copy(zero_v, acc.at[pl.ds(s * WPT + m * ZR, ZR)])
        plsc.subcore_barrier()

        def _group(q, _):
            goff = (b0 + q * K_PIPE) * B_E
            for t in range(K_PIPE):
                off = goff + t * B_E
                for i in range(B_E // 16):
                    dv = dst_v[pl.ds(off + i * 16, 16)]
                    lo = dv - base
                    ok = (lo >= 0) & (lo < CH)
                    idx[t][pl.ds(i * 16, 16)] = jnp.where(ok, lo,
                                                          CH + i * 16 + iota16)
                    sidx[t][pl.ds(i * 16, 16)] = src_v[pl.ds(off + i * 16, 16)]

            @pl.when(gath > 0)
            def _():
                waits = [pltpu.async_copy(table.at[sidx[t]], rows[t], semg[t])
                         for t in range(K_PIPE)]
                for t in range(K_PIPE):
                    waits[t].wait()

            drains = [pltpu.async_copy(rows[t], acc.at[idx[t]], sems[t],
                                       add=True)
                      for t in range(K_PIPE)]
            for t in range(K_PIPE):
                drains[t].wait()
            return 0

        lax.fori_loop(0, (b1 - b0) // K_PIPE, _group, 0)
        plsc.subcore_barrier()
        @pl.when(s < NS - 1)
        def _():
            pltpu.sync_copy(acc.at[pl.ds(s * WPT, WPT)],
                            out.at[c, pl.ds(base + s * WPT, WPT)])

        @pl.when(s == NS - 1)
        def _():
            pltpu.sync_copy(acc.at[pl.ds((NS - 1) * WPT, CH - (NS - 1) * WPT)],
                            out.at[c, pl.ds(base + (NS - 1) * WPT,
                                            CH - (NS - 1) * WPT)])

        plsc.subcore_barrier()


def _make_sc_aggregate(n_dst, nchunk, capb):
    mesh = plsc.VectorSubcoreMesh(core_axis_name="c", subcore_axis_name="s",
                                  num_cores=NC, num_subcores=NS)
    e_tile = capb * B_E
    return pl.kernel(
        functools.partial(_sc_aggregate_body, (n_dst, nchunk, capb)),
        out_type=jax.ShapeDtypeStruct((NC, n_dst, DIM), jnp.float32),
        mesh=mesh,
        scratch_types=[
            pltpu.VMEM((e_tile,), jnp.int32),
            pltpu.VMEM((e_tile,), jnp.int32),
            [pltpu.VMEM((B_E,), jnp.int32) for _ in range(K_PIPE)],
            [pltpu.VMEM((B_E,), jnp.int32) for _ in range(K_PIPE)],
            [pltpu.VMEM((B_E, DIM), jnp.float32) for _ in range(K_PIPE)],
            pltpu.VMEM((ZR, DIM), jnp.float32),
            pltpu.VMEM((16,), jnp.int32),
            pltpu.VMEM_SHARED((ACC_R, DIM), jnp.float32),
            [pltpu.SemaphoreType.DMA for _ in range(K_PIPE)],
            [pltpu.SemaphoreType.DMA for _ in range(K_PIPE)],
        ],
    )


def _mlp_pair_body(x_ref, w_ref, b_ref, o_ref):
    x = x_ref[...]
    for t in range(2):
        h = jnp.dot(x, w_ref[t, 0], preferred_element_type=jnp.float32)
        h = jnp.maximum(h + b_ref[t, 0], 0.0)
        o_ref[t] = jnp.dot(h, w_ref[t, 1],
                           preferred_element_type=jnp.float32) + b_ref[t, 1]


def _mlp_pair(x, w2, b2, blk):
    n = x.shape[0]
    return pl.pallas_call(
        _mlp_pair_body,
        grid=(n // blk,),
        in_specs=[
            pl.BlockSpec((blk, DIM), lambda i: (i, 0)),
            pl.BlockSpec((2, 2, DIM, DIM), lambda i: (0, 0, 0, 0)),
            pl.BlockSpec((2, 2, DIM), lambda i: (0, 0, 0)),
        ],
        out_specs=pl.BlockSpec((2, blk, DIM), lambda i: (0, i, 0)),
        out_shape=jax.ShapeDtypeStruct((2, n, DIM), jnp.float32),
    )(x, w2, b2)


def _gru_body(aggr_ref, invp_ref, invn_ref, h_ref, wih_ref, whh_ref,
              bih_ref, bhh_ref, o_ref):
    xp = aggr_ref[0] * invp_ref[...]
    xn = aggr_ref[1] * invn_ref[...]
    gi = (jnp.dot(xp, wih_ref[0:DIM], preferred_element_type=jnp.float32)
          + jnp.dot(xn, wih_ref[DIM:2 * DIM],
                    preferred_element_type=jnp.float32) + bih_ref[...])
    h = h_ref[...]
    gh = jnp.dot(h, whh_ref[...], preferred_element_type=jnp.float32) + bhh_ref[...]
    r = jax.nn.sigmoid(gi[:, 0:DIM] + gh[:, 0:DIM])
    z = jax.nn.sigmoid(gi[:, DIM:2 * DIM] + gh[:, DIM:2 * DIM])
    nn = jnp.tanh(gi[:, 2 * DIM:] + r * gh[:, 2 * DIM:])
    o_ref[...] = (1.0 - z) * nn + z * h


def _gru(aggr, invp, invn, h, wih_t, whh_t, bih, bhh, blk):
    n = h.shape[0]
    return pl.pallas_call(
        _gru_body,
        grid=(n // blk,),
        in_specs=[
            pl.BlockSpec((2, blk, DIM), lambda i: (0, i, 0)),
            pl.BlockSpec((blk, 1), lambda i: (i, 0)),
            pl.BlockSpec((blk, 1), lambda i: (i, 0)),
            pl.BlockSpec((blk, DIM), lambda i: (i, 0)),
            pl.BlockSpec((2 * DIM, 3 * DIM), lambda i: (0, 0)),
            pl.BlockSpec((DIM, 3 * DIM), lambda i: (0, 0)),
            pl.BlockSpec((3 * DIM,), lambda i: (0,)),
            pl.BlockSpec((3 * DIM,), lambda i: (0,)),
        ],
        out_specs=pl.BlockSpec((blk, DIM), lambda i: (i, 0)),
        out_shape=jax.ShapeDtypeStruct((n, DIM), jnp.float32),
    )(aggr, invp, invn, h, wih_t, whh_t, bih, bhh)


def _pad_edges(x, fill):
    x2 = x.reshape(NS, NE // NS)
    padv = jnp.full((NS, 5120 - NE // NS), fill, jnp.int32)
    return jnp.concatenate([x2, padv], axis=1).reshape(NS * 5120)


def _bucket_edges(src, dst, n_dst, nchunk, capb):
    cid = jnp.minimum(dst // CH, nchunk - 1)
    onek = (cid[None, :] == jnp.arange(nchunk, dtype=jnp.int32)[:, None])
    ranks = jnp.cumsum(onek.astype(jnp.int32), axis=1)
    w = jnp.take_along_axis(ranks, cid[None, :], 0)[0] - 1
    counts = ranks[:, -1]
    lk = ((counts + GRAN - 1) // GRAN) * GRAN
    cum = jnp.concatenate([jnp.zeros((1,), jnp.int32), jnp.cumsum(lk)])
    pt = jnp.maximum(lk // NS, 1)[cid]
    tile = w // pt
    slot = (cum[:nchunk] // NS)[cid] + w % pt
    tgt = tile * (capb * B_E) + slot
    cap = NS * capb * B_E
    packed = jnp.full((cap,), n_dst, jnp.int32).at[tgt].set(
        src * 65536 + dst, unique_indices=True, mode="promise_in_bounds")
    cb = cum // (NS * B_E)
    return packed // 65536, packed % 65536, cb


def kernel(v_init, c_init, mlp_W, mlp_b, gru_Wih, gru_Whh, gru_bih, gru_bhh,
           v_edge_index, c_edge_index, p_edge_index, n_edge_index):
    f32 = jnp.float32
    v_edge_index = v_edge_index.astype(jnp.int32)
    c_edge_index = c_edge_index.astype(jnp.int32)
    p_edge_index = p_edge_index.astype(jnp.int32)
    n_edge_index = n_edge_index.astype(jnp.int32)

    pv = v_edge_index[p_edge_index]
    pc = c_edge_index[p_edge_index]
    nv = v_edge_index[n_edge_index]
    nc = c_edge_index[n_edge_index]

    capb_c = (NE + GRAN - 1) // GRAN * K_PIPE + (4 - 1) * K_PIPE
    capb_v = (NE + GRAN - 1) // GRAN * K_PIPE

    psrc_c, pdst_c, cb_pc = _bucket_edges(pv, pc, C_SIZE, 4, capb_c)
    nsrc_c, ndst_c, cb_nc = _bucket_edges(nv + V_SIZE, nc, C_SIZE, 4, capb_c)
    esrc_c = jnp.concatenate([psrc_c, nsrc_c])
    edst_c = jnp.concatenate([pdst_c, ndst_c])
    esrc_v = jnp.concatenate([_pad_edges(pc, 0),
                              _pad_edges(nc, 0) + C_SIZE])
    edst_v = jnp.concatenate([_pad_edges(pv, V_SIZE),
                              _pad_edges(nv, V_SIZE)])
    cb_pv = cb_nv = jnp.array([0, capb_v], jnp.int32)

    def pack_par(mode, cb0, cb1):
        p = jnp.zeros((16,), jnp.int32)
        p = p.at[0].set(mode)
        p = lax.dynamic_update_slice(p, cb0.astype(jnp.int32), (1,))
        p = lax.dynamic_update_slice(p, cb1.astype(jnp.int32), (6,))
        return p

    par_c_deg = pack_par(0, cb_pc, cb_nc)
    par_c_agg = pack_par(1, cb_pc, cb_nc)
    par_v_deg = pack_par(0, cb_pv, cb_nv)
    par_v_agg = pack_par(1, cb_pv, cb_nv)

    sc_v2c = _make_sc_aggregate(C_SIZE, 4, capb_c)
    sc_c2v = _make_sc_aggregate(V_SIZE, 1, capb_v)

    wih_t = jnp.transpose(gru_Wih, (0, 2, 1)).astype(f32)
    whh_t = jnp.transpose(gru_Whh, (0, 2, 1)).astype(f32)

    init_norm = jnp.sqrt(jnp.asarray(DIM, f32))
    v_emb = jnp.broadcast_to(v_init / init_norm, (V_SIZE, DIM)).astype(f32)
    c_emb = jnp.broadcast_to(c_init / init_norm, (C_SIZE, DIM)).astype(f32)

    inv_pc = inv_nc = inv_pv = inv_nv = None
    for it in range(N_ITER):
        t_v = _mlp_pair(v_emb, mlp_W[0:2], mlp_b[0:2], 2000)
        tvf = t_v.reshape(2 * V_SIZE, DIM)
        if it == 0:
            deg_c = sc_v2c(tvf, esrc_c, edst_c, par_c_deg)
            inv_pc = (1.0 / jnp.maximum(deg_c[0, :, 0], 1.0)).reshape(C_SIZE, 1)
            inv_nc = (1.0 / jnp.maximum(deg_c[1, :, 0], 1.0)).reshape(C_SIZE, 1)
        aggr_c = sc_v2c(tvf, esrc_c, edst_c, par_c_agg)
        c_emb = _gru(aggr_c, inv_pc, inv_nc, c_emb,
                     wih_t[0], whh_t[0], gru_bih[0], gru_bhh[0], 2000)
        t_c = _mlp_pair(c_emb, mlp_W[2:4], mlp_b[2:4], 2000)
        tcf = t_c.reshape(2 * C_SIZE, DIM)
        if it == 0:
            deg_v = sc_c2v(tcf, esrc_v, edst_v, par_v_deg)
            inv_pv = (1.0 / jnp.maximum(deg_v[0, :, 0], 1.0)).reshape(V_SIZE, 1)
            inv_nv = (1.0 / jnp.maximum(deg_v[1, :, 0], 1.0)).reshape(V_SIZE, 1)
        aggr_v = sc_c2v(tcf, esrc_v, edst_v, par_v_agg)
        v_emb = _gru(aggr_v, inv_pv, inv_nv, v_emb,
                     wih_t[1], whh_t[1], gru_bih[1], gru_bhh[1], 2000)
    return v_emb


# speedup vs baseline: 2.3031x; 1.0351x over previous
import functools

import jax
import jax.numpy as jnp
from jax import lax
from jax.experimental import pallas as pl
from jax.experimental.pallas import tpu as pltpu
from jax.experimental.pallas import tpu_sc as plsc

DIM = 128
V_SIZE = 10000
C_SIZE = 40000
N_ITER = 2

NC = 2
NS = 16

NE = 80000
B_E = 64
NB = 80
E_TILE = B_E * NB
E_REAL = NE // NS
NE_PAD = E_TILE * NS
K_PIPE = 4
CH = 10000
ACC_R = 10240
WPT = ACC_R // NS
ZR = 16


def _sc_aggregate_body(nchunk, table, esrc, edst, params, out, src_v, dst_v,
                       sidx, idx, rows, zero_v, par_v, acc, semg, sems):
    c = lax.axis_index("c")
    s = lax.axis_index("s")
    pltpu.sync_copy(params, par_v)
    gath = par_v[pl.ds(0, 16)][0]
    ebase = c * NE_PAD + s * E_TILE
    pltpu.sync_copy(esrc.at[pl.ds(ebase, E_TILE)], src_v)
    pltpu.sync_copy(edst.at[pl.ds(ebase, E_TILE)], dst_v)
    zeros16 = jnp.zeros((16,), jnp.float32)
    ones16 = jnp.ones((16,), jnp.float32)
    iota16 = jnp.arange(16, dtype=jnp.int32)

    def _zrow(i, _):
        for u in range(8):
            zero_v[i, pl.ds(u * 16, 16)] = zeros16
        return 0

    lax.fori_loop(0, ZR, _zrow, 0)

    @pl.when(gath == 0)
    def _():
        def _orow(i, _):
            for t in range(K_PIPE):
                for u in range(8):
                    rows[t][i, pl.ds(u * 16, 16)] = ones16
            return 0

        lax.fori_loop(0, B_E, _orow, 0)

    for k in range(nchunk):
        base = k * CH
        for m in range(WPT // ZR):
            pltpu.sync_copy(zero_v, acc.at[pl.ds(s * WPT + m * ZR, ZR)])
        plsc.subcore_barrier()

        def _group(g, _):
            goff = g * (K_PIPE * B_E)
            for t in range(K_PIPE):
                off = goff + t * B_E
                for i in range(B_E // 16):
                    dv = dst_v[pl.ds(off + i * 16, 16)]
                    lo = dv - base
                    ok = (lo >= 0) & (lo < CH)
                    idx[t][pl.ds(i * 16, 16)] = jnp.where(ok, lo,
                                                          CH + i * 16 + iota16)
                    sidx[t][pl.ds(i * 16, 16)] = src_v[pl.ds(off + i * 16, 16)]

            @pl.when(gath > 0)
            def _():
                waits = [pltpu.async_copy(table.at[sidx[t]], rows[t], semg[t])
                         for t in range(K_PIPE)]
                for t in range(K_PIPE):
                    waits[t].wait()

            drains = [pltpu.async_copy(rows[t], acc.at[idx[t]], sems[t],
                                       add=True)
                      for t in range(K_PIPE)]
            for t in range(K_PIPE):
                drains[t].wait()
            return 0

        lax.fori_loop(0, NB // K_PIPE, _group, 0)
        plsc.subcore_barrier()
        @pl.when(s < NS - 1)
        def _():
            pltpu.sync_copy(acc.at[pl.ds(s * WPT, WPT)],
                            out.at[c, pl.ds(base + s * WPT, WPT)])

        @pl.when(s == NS - 1)
        def _():
            pltpu.sync_copy(acc.at[pl.ds((NS - 1) * WPT, CH - (NS - 1) * WPT)],
                            out.at[c, pl.ds(base + (NS - 1) * WPT,
                                            CH - (NS - 1) * WPT)])

        plsc.subcore_barrier()


def _make_sc_aggregate(n_dst, nchunk):
    mesh = plsc.VectorSubcoreMesh(core_axis_name="c", subcore_axis_name="s",
                                  num_cores=NC, num_subcores=NS)
    return pl.kernel(
        functools.partial(_sc_aggregate_body, nchunk),
        out_type=jax.ShapeDtypeStruct((NC, n_dst, DIM), jnp.float32),
        mesh=mesh,
        scratch_types=[
            pltpu.VMEM((E_TILE,), jnp.int32),
            pltpu.VMEM((E_TILE,), jnp.int32),
            [pltpu.VMEM((B_E,), jnp.int32) for _ in range(K_PIPE)],
            [pltpu.VMEM((B_E,), jnp.int32) for _ in range(K_PIPE)],
            [pltpu.VMEM((B_E, DIM), jnp.float32) for _ in range(K_PIPE)],
            pltpu.VMEM((ZR, DIM), jnp.float32),
            pltpu.VMEM((16,), jnp.int32),
            pltpu.VMEM_SHARED((ACC_R, DIM), jnp.float32),
            [pltpu.SemaphoreType.DMA for _ in range(K_PIPE)],
            [pltpu.SemaphoreType.DMA for _ in range(K_PIPE)],
        ],
    )


def _mlp_pair_body(x_ref, w_ref, b_ref, o_ref):
    x = x_ref[...]
    for t in range(2):
        h = jnp.dot(x, w_ref[t, 0], preferred_element_type=jnp.float32)
        h = jnp.maximum(h + b_ref[t, 0], 0.0)
        o_ref[t] = jnp.dot(h, w_ref[t, 1],
                           preferred_element_type=jnp.float32) + b_ref[t, 1]


def _mlp_pair(x, w2, b2, blk):
    n = x.shape[0]
    return pl.pallas_call(
        _mlp_pair_body,
        grid=(n // blk,),
        in_specs=[
            pl.BlockSpec((blk, DIM), lambda i: (i, 0)),
            pl.BlockSpec((2, 2, DIM, DIM), lambda i: (0, 0, 0, 0)),
            pl.BlockSpec((2, 2, DIM), lambda i: (0, 0, 0)),
        ],
        out_specs=pl.BlockSpec((2, blk, DIM), lambda i: (0, i, 0)),
        out_shape=jax.ShapeDtypeStruct((2, n, DIM), jnp.float32),
    )(x, w2, b2)


def _gru_body(aggr_ref, invp_ref, invn_ref, h_ref, wih_ref, whh_ref,
              bih_ref, bhh_ref, o_ref):
    xp = aggr_ref[0] * invp_ref[...]
    xn = aggr_ref[1] * invn_ref[...]
    gi = (jnp.dot(xp, wih_ref[0:DIM], preferred_element_type=jnp.float32)
          + jnp.dot(xn, wih_ref[DIM:2 * DIM],
                    preferred_element_type=jnp.float32) + bih_ref[...])
    h = h_ref[...]
    gh = jnp.dot(h, whh_ref[...], preferred_element_type=jnp.float32) + bhh_ref[...]
    r = jax.nn.sigmoid(gi[:, 0:DIM] + gh[:, 0:DIM])
    z = jax.nn.sigmoid(gi[:, DIM:2 * DIM] + gh[:, DIM:2 * DIM])
    nn = jnp.tanh(gi[:, 2 * DIM:] + r * gh[:, 2 * DIM:])
    o_ref[...] = (1.0 - z) * nn + z * h


def _gru(aggr, invp, invn, h, wih_t, whh_t, bih, bhh, blk):
    n = h.shape[0]
    return pl.pallas_call(
        _gru_body,
        grid=(n // blk,),
        in_specs=[
            pl.BlockSpec((2, blk, DIM), lambda i: (0, i, 0)),
            pl.BlockSpec((blk, 1), lambda i: (i, 0)),
            pl.BlockSpec((blk, 1), lambda i: (i, 0)),
            pl.BlockSpec((blk, DIM), lambda i: (i, 0)),
            pl.BlockSpec((2 * DIM, 3 * DIM), lambda i: (0, 0)),
            pl.BlockSpec((DIM, 3 * DIM), lambda i: (0, 0)),
            pl.BlockSpec((3 * DIM,), lambda i: (0,)),
            pl.BlockSpec((3 * DIM,), lambda i: (0,)),
        ],
        out_specs=pl.BlockSpec((blk, DIM), lambda i: (i, 0)),
        out_shape=jax.ShapeDtypeStruct((n, DIM), jnp.float32),
    )(aggr, invp, invn, h, wih_t, whh_t, bih, bhh)


def _pad_edges(x, fill):
    x2 = x.reshape(NS, E_REAL)
    padv = jnp.full((NS, E_TILE - E_REAL), fill, jnp.int32)
    return jnp.concatenate([x2, padv], axis=1).reshape(NE_PAD)


def kernel(v_init, c_init, mlp_W, mlp_b, gru_Wih, gru_Whh, gru_bih, gru_bhh,
           v_edge_index, c_edge_index, p_edge_index, n_edge_index):
    f32 = jnp.float32
    v_edge_index = v_edge_index.astype(jnp.int32)
    c_edge_index = c_edge_index.astype(jnp.int32)
    p_edge_index = p_edge_index.astype(jnp.int32)
    n_edge_index = n_edge_index.astype(jnp.int32)

    pv = v_edge_index[p_edge_index]
    pc = c_edge_index[p_edge_index]
    nv = v_edge_index[n_edge_index]
    nc = c_edge_index[n_edge_index]

    edst_c = jnp.concatenate([_pad_edges(pc, C_SIZE), _pad_edges(nc, C_SIZE)])
    esrc_c = jnp.concatenate([_pad_edges(pv, 0),
                              _pad_edges(nv, 0) + V_SIZE])
    edst_v = jnp.concatenate([_pad_edges(pv, V_SIZE), _pad_edges(nv, V_SIZE)])
    esrc_v = jnp.concatenate([_pad_edges(pc, 0),
                              _pad_edges(nc, 0) + C_SIZE])

    sc_v2c = _make_sc_aggregate(C_SIZE, C_SIZE // CH)
    sc_c2v = _make_sc_aggregate(V_SIZE, V_SIZE // CH)
    par_deg = jnp.zeros((16,), jnp.int32)
    par_agg = jnp.ones((16,), jnp.int32)

    wih_t = jnp.transpose(gru_Wih, (0, 2, 1)).astype(f32)
    whh_t = jnp.transpose(gru_Whh, (0, 2, 1)).astype(f32)

    init_norm = jnp.sqrt(jnp.asarray(DIM, f32))
    v_emb = jnp.broadcast_to(v_init / init_norm, (V_SIZE, DIM)).astype(f32)
    c_emb = jnp.broadcast_to(c_init / init_norm, (C_SIZE, DIM)).astype(f32)

    inv_pc = inv_nc = inv_pv = inv_nv = None
    for it in range(N_ITER):
        t_v = _mlp_pair(v_emb, mlp_W[0:2], mlp_b[0:2], 2000)
        tvf = t_v.reshape(2 * V_SIZE, DIM)
        if it == 0:
            deg_c = sc_v2c(tvf, esrc_c, edst_c, par_deg)
            inv_pc = (1.0 / jnp.maximum(deg_c[0, :, 0], 1.0)).reshape(C_SIZE, 1)
            inv_nc = (1.0 / jnp.maximum(deg_c[1, :, 0], 1.0)).reshape(C_SIZE, 1)
        aggr_c = sc_v2c(tvf, esrc_c, edst_c, par_agg)
        c_emb = _gru(aggr_c, inv_pc, inv_nc, c_emb,
                     wih_t[0], whh_t[0], gru_bih[0], gru_bhh[0], 2000)
        t_c = _mlp_pair(c_emb, mlp_W[2:4], mlp_b[2:4], 2000)
        tcf = t_c.reshape(2 * C_SIZE, DIM)
        if it == 0:
            deg_v = sc_c2v(tcf, esrc_v, edst_v, par_deg)
            inv_pv = (1.0 / jnp.maximum(deg_v[0, :, 0], 1.0)).reshape(V_SIZE, 1)
            inv_nv = (1.0 / jnp.maximum(deg_v[1, :, 0], 1.0)).reshape(V_SIZE, 1)
        aggr_v = sc_c2v(tcf, esrc_v, edst_v, par_agg)
        v_emb = _gru(aggr_v, inv_pv, inv_nv, v_emb,
                     wih_t[1], whh_t[1], gru_bih[1], gru_bhh[1], 2000)
    return v_emb
